# baseline (speedup 1.0000x reference)
import functools

import jax
import jax.numpy as jnp
from jax.experimental import pallas as pl

N_PTS = 1048576
EMB = 16
HID = 64
SD = 32
BLK = 4096

_PREC = jax.lax.Precision.HIGHEST


def _mlp_body(emb_ref, dx_ref, dy_ref, dz_ref, pos_ref,
              w0e_ref, w0p_ref, w1t_ref, wft_ref, w1_ref, w0eb_ref, w0pb_ref,
              b0_ref, b1_ref, bf_ref, wfr_ref,
              sdf_ref, gx_ref, gy_ref, gz_ref):
    emb = emb_ref[...]
    pos = pos_ref[...]
    z0 = 30.0 * (jnp.dot(emb, w0e_ref[...], precision=_PREC)
                 + jnp.dot(pos, w0p_ref[...], precision=_PREC)
                 + b0_ref[...])
    h0 = jnp.sin(z0)
    c0 = jnp.cos(z0)
    z1 = 30.0 * (jnp.dot(h0, w1t_ref[...], precision=_PREC) + b1_ref[...])
    h1 = jnp.sin(z1)
    c1 = jnp.cos(z1)
    sdf_ref[...] = jnp.dot(h1, wft_ref[...], precision=_PREC) + bf_ref[...]
    gz1 = wfr_ref[...] * c1
    gh0 = 30.0 * jnp.dot(gz1, w1_ref[...], precision=_PREC)
    gz0 = gh0 * c0
    ge = 30.0 * jnp.dot(gz0, w0eb_ref[...], precision=_PREC)
    gp = 30.0 * jnp.dot(gz0, w0pb_ref[...], precision=_PREC)
    gx_ref[...] = gp[:, 0:1] + jnp.sum(ge * dx_ref[...], axis=1, keepdims=True)
    gy_ref[...] = gp[:, 1:2] + jnp.sum(ge * dy_ref[...], axis=1, keepdims=True)
    gz_ref[...] = gp[:, 2:3] + jnp.sum(ge * dz_ref[...], axis=1, keepdims=True)


def _mlp_call(emb, Dx, Dy, Dz, pos, W0, b0, W1, b1, Wf, bf):
    nblk = N_PTS // BLK
    bspec = lambda w: pl.BlockSpec((BLK, w), lambda i: (i, 0))
    wspec = lambda a: pl.BlockSpec(a.shape, lambda i: (0,) * a.ndim)
    weights = (W0[:, :EMB].T, W0[:, EMB:].T, W1.T, Wf.T, W1,
               W0[:, :EMB], W0[:, EMB:],
               b0[None, :], b1[None, :], bf[None, :], Wf)
    out = pl.pallas_call(
        _mlp_body,
        grid=(nblk,),
        in_specs=[bspec(EMB), bspec(EMB), bspec(EMB), bspec(EMB), bspec(3)]
                 + [wspec(w) for w in weights],
        out_specs=[bspec(1), bspec(1), bspec(1), bspec(1)],
        out_shape=[jax.ShapeDtypeStruct((N_PTS, 1), jnp.float32)] * 4,
    )(emb, Dx, Dy, Dz, pos, *weights)
    sdf, gx, gy, gz = out
    return sdf, jnp.concatenate([gx, gy, gz], axis=1)


def _interp_parts(grid, pos, bmin, bmax):
    R = grid.shape[0] - 1
    scale = R / (bmax - bmin)
    u = (pos - bmin) * scale
    i0 = jnp.floor(u).astype(jnp.int32)
    f = u - i0.astype(jnp.float32)
    fx, fy, fz = f[:, 0:1], f[:, 1:2], f[:, 2:3]
    emb = Dx = Dy = Dz = 0.0
    for a in (0, 1):
        wa, da = (fx, 1.0) if a else (1.0 - fx, -1.0)
        for b in (0, 1):
            wb, db = (fy, 1.0) if b else (1.0 - fy, -1.0)
            for c in (0, 1):
                wc, dc = (fz, 1.0) if c else (1.0 - fz, -1.0)
                C = grid[i0[:, 0] + a, i0[:, 1] + b, i0[:, 2] + c]
                emb = emb + wa * wb * wc * C
                Dx = Dx + da * wb * wc * C
                Dy = Dy + wa * db * wc * C
                Dz = Dz + wa * wb * dc * C
    return emb, Dx * scale, Dy * scale, Dz * scale


def kernel(positions, grid_main, grid_empty, occupancy, W0, b0, W1, b1, Wf, bf):
    em, Dxm, Dym, Dzm = _interp_parts(grid_main, positions, 0.0, 1.0)
    ee, Dxe, Dye, Dze = _interp_parts(grid_empty, positions, -0.1, 1.1)
    cell = jnp.floor(positions * SD).astype(jnp.int32)
    mask = occupancy[cell[:, 0], cell[:, 1], cell[:, 2]][:, None]
    emb = jnp.where(mask, em, ee)
    Dx = jnp.where(mask, Dxm, Dxe)
    Dy = jnp.where(mask, Dym, Dye)
    Dz = jnp.where(mask, Dzm, Dze)
    return _mlp_call(emb, Dx, Dy, Dz, positions, W0, b0, W1, b1, Wf, bf)


# speedup vs baseline: 5.4191x; 5.4191x over previous
import functools

import jax
import jax.numpy as jnp
from jax import lax
from jax.experimental import pallas as pl
from jax.experimental.pallas import tpu as pltpu
from jax.experimental.pallas import tpu_sc as plsc

N_PTS = 1048576
EMB = 16
HID = 64
SD = 32
GRID_R = 128
V_MAIN = 129 * 129 * 129
V_EMPTY = 125
ES = 10.0 / 3.0

BLK = 4096

NC, NS, L = 2, 16, 16
NW = NC * NS
P = 128
NT = N_PTS // NW
CHUNKS = NT // P
G = P // L

_PREC = jax.lax.Precision.HIGHEST


def _sc_body(posT_hbm, table_hbm, occp_hbm,
             emb_hbm, dx_hbm, dy_hbm, dz_hbm,
             pos_s, idx_s, feats_s, fs_s, out_s, occ_s,
             sp0, sp1, sg0, sg1, so0, so1):
    wid = lax.axis_index("s") * NC + lax.axis_index("c")
    base = wid * NT
    lane = lax.iota(jnp.int32, L)

    pltpu.sync_copy(occp_hbm, occ_s)

    def fire_pos(kk, b):
        pltpu.async_copy(posT_hbm.at[:, pl.ds(base + kk * P, P)],
                         pos_s.at[b], (sp0, sp1)[b])

    def wait_pos(b):
        pltpu.make_async_copy(posT_hbm.at[:, pl.ds(0, P)],
                              pos_s.at[b], (sp0, sp1)[b]).wait()

    def build(kk, b):
        def grp(g, _):
            px = pos_s[b, 0, pl.ds(g * L, L)]
            py = pos_s[b, 1, pl.ds(g * L, L)]
            pz = pos_s[b, 2, pl.ds(g * L, L)]
            cx = (px * float(SD)).astype(jnp.int32)
            cy = (py * float(SD)).astype(jnp.int32)
            cz = (pz * float(SD)).astype(jnp.int32)
            oflat = (cx * SD + cy) * SD + cz
            word = plsc.load_gather(occ_s, [lax.shift_right_logical(oflat, 5)])
            bit = lax.shift_right_logical(word, jnp.bitwise_and(oflat, 31))
            m = jnp.bitwise_and(bit, 1) > 0
            umx = px * float(GRID_R)
            umy = py * float(GRID_R)
            umz = pz * float(GRID_R)
            imx = umx.astype(jnp.int32)
            imy = umy.astype(jnp.int32)
            imz = umz.astype(jnp.int32)
            uex = (px + 0.1) * ES
            uey = (py + 0.1) * ES
            uez = (pz + 0.1) * ES
            iex = uex.astype(jnp.int32)
            iey = uey.astype(jnp.int32)
            iez = uez.astype(jnp.int32)
            fx = jnp.where(m, umx - imx.astype(jnp.float32),
                           uex - iex.astype(jnp.float32))
            fy = jnp.where(m, umy - imy.astype(jnp.float32),
                           uey - iey.astype(jnp.float32))
            fz = jnp.where(m, umz - imz.astype(jnp.float32),
                           uez - iez.astype(jnp.float32))
            sc = jnp.where(m, float(GRID_R), ES)
            flat_m = (imx * 129 + imy) * 129 + imz
            flat_e = (iex * 5 + iey) * 5 + iez + V_MAIN
            b00 = jnp.where(m, flat_m, flat_e)
            s1 = jnp.where(m, 129 * 129, 25)
            s2 = jnp.where(m, 129, 5)
            b10 = b00 + s1
            b01 = b00 + s2
            b11 = b10 + s2
            o = g * L
            idx_s[b, 0, pl.ds(o, L)] = b00
            idx_s[b, 1, pl.ds(o, L)] = b00 + 1
            idx_s[b, 2, pl.ds(o, L)] = b01
            idx_s[b, 3, pl.ds(o, L)] = b01 + 1
            idx_s[b, 4, pl.ds(o, L)] = b10
            idx_s[b, 5, pl.ds(o, L)] = b10 + 1
            idx_s[b, 6, pl.ds(o, L)] = b11
            idx_s[b, 7, pl.ds(o, L)] = b11 + 1
            fs_s[b, 0, pl.ds(o, L)] = fx
            fs_s[b, 1, pl.ds(o, L)] = fy
            fs_s[b, 2, pl.ds(o, L)] = fz
            fs_s[b, 3, pl.ds(o, L)] = sc
            return 0

        lax.fori_loop(0, G, grp, 0, unroll=False)
        for jc in range(8):
            pltpu.async_copy(table_hbm.at[idx_s.at[b, jc]],
                             feats_s.at[b, pl.ds(jc * P, P)],
                             (sg0, sg1)[b])

    def wait_gather(b):
        for jc in range(8):
            pltpu.make_async_copy(table_hbm.at[idx_s.at[b, jc]],
                                  feats_s.at[b, pl.ds(jc * P, P)],
                                  (sg0, sg1)[b]).wait()

    def out_refs(kk):
        sl = pl.ds(base + kk * P, P)
        return (emb_hbm.at[sl], dx_hbm.at[sl], dy_hbm.at[sl], dz_hbm.at[sl])

    def combine(kk, b):
        def grp(g, _):
            o = g * L
            fx = fs_s[b, 0, pl.ds(o, L)]
            fy = fs_s[b, 1, pl.ds(o, L)]
            fz = fs_s[b, 2, pl.ds(o, L)]
            sc = fs_s[b, 3, pl.ds(o, L)]
            gx1 = 1.0 - fx
            gy1 = 1.0 - fy
            w00 = gx1 * gy1
            w01 = gx1 * fy
            w10 = fx * gy1
            w11 = fx * fy
            rows = o + lane
            r = [rows + j * P for j in range(8)]
            feats = feats_s.at[b]
            out = out_s.at[b]
            for d in range(EMB):
                col = jnp.full((L,), d, jnp.int32)
                c000 = plsc.load_gather(feats, [r[0], col])
                c001 = plsc.load_gather(feats, [r[1], col])
                c010 = plsc.load_gather(feats, [r[2], col])
                c011 = plsc.load_gather(feats, [r[3], col])
                c100 = plsc.load_gather(feats, [r[4], col])
                c101 = plsc.load_gather(feats, [r[5], col])
                c110 = plsc.load_gather(feats, [r[6], col])
                c111 = plsc.load_gather(feats, [r[7], col])
                t00 = c001 - c000
                t01 = c011 - c010
                t10 = c101 - c100
                t11 = c111 - c110
                dz = ((w00 * t00 + w01 * t01) + (w10 * t10 + w11 * t11)) * sc
                cz00 = c000 + fz * t00
                cz01 = c010 + fz * t01
                cz10 = c100 + fz * t10
                cz11 = c110 + fz * t11
                ty0 = cz01 - cz00
                ty1 = cz11 - cz10
                dy = (gx1 * ty0 + fx * ty1) * sc
                u0 = cz00 + fy * ty0
                u1 = cz10 + fy * ty1
                tx = u1 - u0
                emb = u0 + fx * tx
                dx = tx * sc
                plsc.store_scatter(out.at[0], [rows, col], emb)
                plsc.store_scatter(out.at[1], [rows, col], dx)
                plsc.store_scatter(out.at[2], [rows, col], dy)
                plsc.store_scatter(out.at[3], [rows, col], dz)
            return 0

        lax.fori_loop(0, G, grp, 0, unroll=False)
        so = (so0, so1)[b]
        for i, oref in enumerate(out_refs(kk)):
            pltpu.async_copy(out_s.at[b, i], oref, so)

    def wait_out(b):
        so = (so0, so1)[b]
        for i, oref in enumerate(out_refs(0)):
            pltpu.make_async_copy(out_s.at[b, i], oref, so).wait()

    fire_pos(0, 0)
    fire_pos(1, 1)
    wait_pos(0)
    build(0, 0)

    def step(i, _):
        for half in range(2):
            kk = 2 * i + half
            b = half
            nb = 1 - half

            @pl.when(kk + 2 < CHUNKS)
            def _():
                fire_pos(kk + 2, b)

            @pl.when(kk + 1 < CHUNKS)
            def _():
                wait_pos(nb)
                build(kk + 1, nb)

            @pl.when(kk >= 2)
            def _():
                wait_out(b)

            wait_gather(b)
            combine(kk, b)
        return 0

    lax.fori_loop(0, CHUNKS // 2, step, 0, unroll=False)
    wait_out(0)
    wait_out(1)


def _interp_sc(positions, grid_main, grid_empty, occupancy):
    posT = positions.T
    table = jnp.concatenate(
        [grid_main.reshape(V_MAIN, EMB), grid_empty.reshape(V_EMPTY, EMB)],
        axis=0)
    occ_u = occupancy.reshape(SD * SD * SD // 32, 32).astype(jnp.uint32)
    shifts = jnp.arange(32, dtype=jnp.uint32)
    occp = lax.bitcast_convert_type(
        (occ_u << shifts).sum(axis=1, dtype=jnp.uint32), jnp.int32)

    mesh = plsc.VectorSubcoreMesh(core_axis_name="c", subcore_axis_name="s",
                                  num_cores=NC, num_subcores=NS)
    f32 = jnp.float32
    out = jax.ShapeDtypeStruct((N_PTS, EMB), f32)
    run = pl.kernel(
        _sc_body,
        out_type=(out, out, out, out),
        mesh=mesh,
        compiler_params=pltpu.CompilerParams(needs_layout_passes=False,
                                             use_tc_tiling_on_sc=False),
        scratch_types=(
            pltpu.VMEM((2, 3, P), f32),
            pltpu.VMEM((2, 8, P), jnp.int32),
            pltpu.VMEM((2, 8 * P, EMB), f32),
            pltpu.VMEM((2, 4, P), f32),
            pltpu.VMEM((2, 4, P, EMB), f32),
            pltpu.VMEM((1024,), jnp.int32),
            pltpu.SemaphoreType.DMA,
            pltpu.SemaphoreType.DMA,
            pltpu.SemaphoreType.DMA,
            pltpu.SemaphoreType.DMA,
            pltpu.SemaphoreType.DMA,
            pltpu.SemaphoreType.DMA,
        ),
    )
    return run(posT, table, occp)


def _mlp_body(emb_ref, dx_ref, dy_ref, dz_ref, pos_ref,
              w0e_ref, w0p_ref, w1t_ref, wft_ref, w1_ref, w0eb_ref, w0pb_ref,
              b0_ref, b1_ref, bf_ref, wfr_ref,
              sdf_ref, gx_ref, gy_ref, gz_ref):
    emb = emb_ref[...]
    pos = pos_ref[...]
    z0 = 30.0 * (jnp.dot(emb, w0e_ref[...], precision=_PREC)
                 + jnp.dot(pos, w0p_ref[...], precision=_PREC)
                 + b0_ref[...])
    h0 = jnp.sin(z0)
    c0 = jnp.cos(z0)
    z1 = 30.0 * (jnp.dot(h0, w1t_ref[...], precision=_PREC) + b1_ref[...])
    h1 = jnp.sin(z1)
    c1 = jnp.cos(z1)
    sdf_ref[...] = jnp.dot(h1, wft_ref[...], precision=_PREC) + bf_ref[...]
    gz1 = wfr_ref[...] * c1
    gh0 = 30.0 * jnp.dot(gz1, w1_ref[...], precision=_PREC)
    gz0 = gh0 * c0
    ge = 30.0 * jnp.dot(gz0, w0eb_ref[...], precision=_PREC)
    gp = 30.0 * jnp.dot(gz0, w0pb_ref[...], precision=_PREC)
    gx_ref[...] = gp[:, 0:1] + jnp.sum(ge * dx_ref[...], axis=1, keepdims=True)
    gy_ref[...] = gp[:, 1:2] + jnp.sum(ge * dy_ref[...], axis=1, keepdims=True)
    gz_ref[...] = gp[:, 2:3] + jnp.sum(ge * dz_ref[...], axis=1, keepdims=True)


def _mlp_call(emb, Dx, Dy, Dz, pos, W0, b0, W1, b1, Wf, bf):
    nblk = N_PTS // BLK
    bspec = lambda w: pl.BlockSpec((BLK, w), lambda i: (i, 0))
    wspec = lambda a: pl.BlockSpec(a.shape, lambda i: (0,) * a.ndim)
    weights = (W0[:, :EMB].T, W0[:, EMB:].T, W1.T, Wf.T, W1,
               W0[:, :EMB], W0[:, EMB:],
               b0[None, :], b1[None, :], bf[None, :], Wf)
    out = pl.pallas_call(
        _mlp_body,
        grid=(nblk,),
        in_specs=[bspec(EMB), bspec(EMB), bspec(EMB), bspec(EMB), bspec(3)]
                 + [wspec(w) for w in weights],
        out_specs=[bspec(1), bspec(1), bspec(1), bspec(1)],
        out_shape=[jax.ShapeDtypeStruct((N_PTS, 1), jnp.float32)] * 4,
    )(emb, Dx, Dy, Dz, pos, *weights)
    sdf, gx, gy, gz = out
    return sdf, jnp.concatenate([gx, gy, gz], axis=1)


def kernel(positions, grid_main, grid_empty, occupancy, W0, b0, W1, b1, Wf, bf):
    emb, Dx, Dy, Dz = _interp_sc(positions, grid_main, grid_empty, occupancy)
    return _mlp_call(emb, Dx, Dy, Dz, positions, W0, b0, W1, b1, Wf, bf)
